# baseline (speedup 1.0000x reference)
import functools

import jax
import jax.numpy as jnp
from jax.experimental import pallas as pl
from jax.experimental.pallas import tpu as pltpu


def _round_up(x, m):
    return -(-x // m) * m


def _sigmoid(x):
    return 0.5 * (jnp.tanh(0.5 * x) + 1.0)


def _lstm_kernel(vpre_ref, sh_ref, emb_ref, wih_ref, whh_ref, b_ref, wfc_ref,
                 bfc_ref, out_ref, xg_ref, xt_ref, h_sc, c_sc, *,
                 seq_len, chunk_steps, bb, blk_rows):
    E = emb_ref.shape[1]
    H = whh_ref.shape[0]
    c_idx = pl.program_id(1)
    rows_chunk = chunk_steps * bb
    n_blocks = rows_chunk // blk_rows
    tok0 = pl.program_id(0) * (seq_len * bb) + c_idx * rows_chunk
    row_iota = jax.lax.broadcasted_iota(jnp.int32, (8, E), 0)

    @pl.when(c_idx == 0)
    def _():
        h_sc[...] = jnp.zeros_like(h_sc)
        c_sc[...] = jnp.zeros_like(c_sc)

    def p1_body(blk, _):
        tbase = tok0 + blk * blk_rows
        for run in range(blk_rows // 8):
            acc = jnp.zeros((8, E), jnp.float32)
            for k in range(8):
                mi = run * 8 + k
                vp = vpre_ref[tbase + mi]
                c8 = pl.multiple_of(vp << 3, 8)
                chunk = emb_ref[pl.ds(c8, 8), :]
                rolled = pltpu.roll(chunk, sh_ref[tbase + mi], axis=0)
                acc = jnp.where(row_iota == k, rolled, acc)
            xt_ref[run * 8:(run + 1) * 8, :] = acc
        xgb = (jnp.dot(xt_ref[...].astype(jnp.bfloat16), wih_ref[...],
                       preferred_element_type=jnp.float32) + b_ref[...])
        r0 = pl.multiple_of(blk * blk_rows, blk_rows)
        xg_ref[pl.ds(r0, blk_rows), :] = xgb.astype(jnp.bfloat16)
        return 0

    jax.lax.fori_loop(0, n_blocks, p1_body, 0)

    def step(t, carry):
        h, c = carry
        r0 = pl.multiple_of(t * bb, bb)
        gates = (xg_ref[pl.ds(r0, bb), :].astype(jnp.float32)
                 + jnp.dot(h.astype(jnp.bfloat16), whh_ref[...],
                           preferred_element_type=jnp.float32))
        i_g = _sigmoid(gates[:, 0 * H:1 * H])
        f_g = _sigmoid(gates[:, 1 * H:2 * H])
        g_g = jnp.tanh(gates[:, 2 * H:3 * H])
        o_g = _sigmoid(gates[:, 3 * H:4 * H])
        c_new = f_g * c + i_g * g_g
        h_new = o_g * jnp.tanh(c_new)
        return h_new, c_new

    h, c = jax.lax.fori_loop(0, chunk_steps, step, (h_sc[...], c_sc[...]))
    h_sc[...] = h
    c_sc[...] = c

    @pl.when(c_idx == pl.num_programs(1) - 1)
    def _():
        logits = (jnp.dot(h.astype(jnp.bfloat16), wfc_ref[...],
                          preferred_element_type=jnp.float32) + bfc_ref[...])
        out_ref[...] = _sigmoid(logits)


def kernel(token_ids, embedding, w_ih, w_hh, b, w_fc, b_fc):
    B, T = token_ids.shape
    V, E = embedding.shape
    H = w_hh.shape[0]
    O = w_fc.shape[1]

    n_cores = 2
    bb = B // n_cores
    n_chunks = 2 if T % 2 == 0 else 1
    chunk_steps = T // n_chunks
    blk_rows = 128
    while (chunk_steps * bb) % blk_rows:
        blk_rows //= 2

    ids = token_ids.astype(jnp.int32).reshape(n_cores, bb, T).transpose(0, 2, 1)
    vpre = (ids >> 3).reshape(-1)
    sh = ((jnp.arange(bb, dtype=jnp.int32)[None, None, :] - ids) & 7).reshape(-1)

    emb = embedding
    if V % 8:
        emb = jnp.pad(emb, ((0, _round_up(V, 8) - V), (0, 0)))

    wih = w_ih.astype(jnp.bfloat16)
    whh = w_hh.astype(jnp.bfloat16)
    O_pad = max(128, _round_up(O, 128))
    wfc = jnp.pad(w_fc, ((0, 0), (0, O_pad - O))).astype(jnp.bfloat16)
    bfc = jnp.pad(b_fc, ((0, 0), (0, O_pad - O)))

    kfn = functools.partial(_lstm_kernel, seq_len=T, chunk_steps=chunk_steps,
                            bb=bb, blk_rows=blk_rows)

    out = pl.pallas_call(
        kfn,
        out_shape=jax.ShapeDtypeStruct((B, O_pad), jnp.float32),
        grid_spec=pltpu.PrefetchScalarGridSpec(
            num_scalar_prefetch=2,
            grid=(n_cores, n_chunks),
            in_specs=[
                pl.BlockSpec(emb.shape, lambda i, c, vp, sh: (0, 0)),
                pl.BlockSpec(wih.shape, lambda i, c, vp, sh: (0, 0)),
                pl.BlockSpec(whh.shape, lambda i, c, vp, sh: (0, 0)),
                pl.BlockSpec(b.shape, lambda i, c, vp, sh: (0, 0)),
                pl.BlockSpec(wfc.shape, lambda i, c, vp, sh: (0, 0)),
                pl.BlockSpec(bfc.shape, lambda i, c, vp, sh: (0, 0)),
            ],
            out_specs=pl.BlockSpec((bb, O_pad), lambda i, c, vp, sh: (i, 0)),
            scratch_shapes=[
                pltpu.VMEM((chunk_steps * bb, 4 * H), jnp.bfloat16),
                pltpu.VMEM((blk_rows, E), jnp.float32),
                pltpu.VMEM((bb, H), jnp.float32),
                pltpu.VMEM((bb, H), jnp.float32),
            ],
        ),
        compiler_params=pltpu.CompilerParams(
            dimension_semantics=("parallel", "arbitrary"),
            vmem_limit_bytes=52 << 20),
    )(vpre, sh, emb, wih, whh, b, wfc, bfc)

    return out[:, :O]


# speedup vs baseline: 1.1838x; 1.1838x over previous
import functools

import jax
import jax.numpy as jnp
from jax.experimental import pallas as pl
from jax.experimental.pallas import tpu as pltpu


def _round_up(x, m):
    return -(-x // m) * m


def _sigmoid(x):
    return 0.5 * (jnp.tanh(0.5 * x) + 1.0)


def _lstm_kernel(vpre_ref, sh_ref, emb_ref, wih_ref, whh_ref, b_ref, wfc_ref,
                 bfc_ref, out_ref, xg_ref, xt_ref, h_sc, c_sc, *,
                 chunk_steps, bb, blk_rows):
    E = emb_ref.shape[1]
    H = whh_ref.shape[0]
    c_idx = pl.program_id(0)
    rows_chunk = chunk_steps * bb
    n_blocks = rows_chunk // blk_rows
    tok0 = c_idx * rows_chunk
    row_iota = jax.lax.broadcasted_iota(jnp.int32, (8, E), 0)

    @pl.when(c_idx == 0)
    def _():
        h_sc[...] = jnp.zeros_like(h_sc)
        c_sc[...] = jnp.zeros_like(c_sc)

    def p1_body(blk, _):
        tbase = tok0 + blk * blk_rows
        for run in range(blk_rows // 8):
            acc = jnp.zeros((8, E), jnp.float32)
            for k in range(8):
                mi = run * 8 + k
                vp = vpre_ref[tbase + mi]
                c8 = pl.multiple_of(vp << 3, 8)
                chunk = emb_ref[pl.ds(c8, 8), :]
                rolled = pltpu.roll(chunk, sh_ref[tbase + mi], axis=0)
                acc = jnp.where(row_iota == k, rolled, acc)
            xt_ref[run * 8:(run + 1) * 8, :] = acc
        xgb = (jnp.dot(xt_ref[...].astype(jnp.bfloat16), wih_ref[...],
                       preferred_element_type=jnp.float32) + b_ref[...])
        r0 = pl.multiple_of(blk * blk_rows, blk_rows)
        xg_ref[pl.ds(r0, blk_rows), :] = xgb.astype(jnp.bfloat16)
        return 0

    jax.lax.fori_loop(0, n_blocks, p1_body, 0)

    def step(t, carry):
        h, c = carry
        r0 = pl.multiple_of(t * bb, bb)
        gates = (xg_ref[pl.ds(r0, bb), :].astype(jnp.float32)
                 + jnp.dot(h.astype(jnp.bfloat16), whh_ref[...],
                           preferred_element_type=jnp.float32))
        i_g = _sigmoid(gates[:, 0 * H:1 * H])
        f_g = _sigmoid(gates[:, 1 * H:2 * H])
        g_g = jnp.tanh(gates[:, 2 * H:3 * H])
        o_g = _sigmoid(gates[:, 3 * H:4 * H])
        c_new = f_g * c + i_g * g_g
        h_new = o_g * jnp.tanh(c_new)
        return h_new, c_new

    h, c = jax.lax.fori_loop(0, chunk_steps, step, (h_sc[...], c_sc[...]))
    h_sc[...] = h
    c_sc[...] = c

    @pl.when(c_idx == pl.num_programs(0) - 1)
    def _():
        logits = (jnp.dot(h.astype(jnp.bfloat16), wfc_ref[...],
                          preferred_element_type=jnp.float32) + bfc_ref[...])
        out_ref[...] = _sigmoid(logits)


def kernel(token_ids, embedding, w_ih, w_hh, b, w_fc, b_fc):
    B, T = token_ids.shape
    V, E = embedding.shape
    H = w_hh.shape[0]
    O = w_fc.shape[1]

    bb = B
    n_chunks = 4
    while T % n_chunks:
        n_chunks //= 2
    chunk_steps = T // n_chunks
    blk_rows = 128
    while (chunk_steps * bb) % blk_rows:
        blk_rows //= 2

    ids = token_ids.astype(jnp.int32).T
    vpre = (ids >> 3).reshape(-1)
    sh = ((jnp.arange(bb, dtype=jnp.int32)[None, :] - ids) & 7).reshape(-1)

    emb = embedding
    if V % 8:
        emb = jnp.pad(emb, ((0, _round_up(V, 8) - V), (0, 0)))

    wih = w_ih.astype(jnp.bfloat16)
    whh = w_hh.astype(jnp.bfloat16)
    O_pad = max(128, _round_up(O, 128))
    wfc = jnp.pad(w_fc, ((0, 0), (0, O_pad - O))).astype(jnp.bfloat16)
    bfc = jnp.pad(b_fc, ((0, 0), (0, O_pad - O)))

    kfn = functools.partial(_lstm_kernel, chunk_steps=chunk_steps,
                            bb=bb, blk_rows=blk_rows)

    out = pl.pallas_call(
        kfn,
        out_shape=jax.ShapeDtypeStruct((B, O_pad), jnp.float32),
        grid_spec=pltpu.PrefetchScalarGridSpec(
            num_scalar_prefetch=2,
            grid=(n_chunks,),
            in_specs=[
                pl.BlockSpec(emb.shape, lambda c, vp, sh: (0, 0)),
                pl.BlockSpec(wih.shape, lambda c, vp, sh: (0, 0)),
                pl.BlockSpec(whh.shape, lambda c, vp, sh: (0, 0)),
                pl.BlockSpec(b.shape, lambda c, vp, sh: (0, 0)),
                pl.BlockSpec(wfc.shape, lambda c, vp, sh: (0, 0)),
                pl.BlockSpec(bfc.shape, lambda c, vp, sh: (0, 0)),
            ],
            out_specs=pl.BlockSpec((bb, O_pad), lambda c, vp, sh: (0, 0)),
            scratch_shapes=[
                pltpu.VMEM((chunk_steps * bb, 4 * H), jnp.bfloat16),
                pltpu.VMEM((blk_rows, E), jnp.float32),
                pltpu.VMEM((bb, H), jnp.float32),
                pltpu.VMEM((bb, H), jnp.float32),
            ],
        ),
        compiler_params=pltpu.CompilerParams(
            dimension_semantics=("arbitrary",),
            vmem_limit_bytes=52 << 20),
    )(vpre, sh, emb, wih, whh, b, wfc, bfc)

    return out[:, :O]
